# Initial kernel scaffold; baseline (speedup 1.0000x reference)
import jax
import jax.numpy as jnp
from jax.experimental import pallas as pl


def kernel(hidden_states, gate_weight, e_score_correction_bias, w13, w2, shared_gate_up, shared_down):
    raise NotImplementedError("write your pallas kernel here")


# speedup vs baseline: 1.9275x; 1.9275x over previous
import functools

import jax
import jax.numpy as jnp
from jax import lax
from jax.experimental import pallas as pl
from jax.experimental.pallas import tpu as pltpu

T = 2048
D = 1024
E = 8
TOPK = 2
DFF = 512
NG = 4
TG = 2
NSH = 2
RSF = 2.5


def _rank_lt(vals, k):
    n = vals.shape[-1]
    a = vals[..., :, None]
    b = vals[..., None, :]
    idx = lax.broadcasted_iota(jnp.int32, (n, n), 0)
    jdx = lax.broadcasted_iota(jnp.int32, (n, n), 1)
    beats = (b > a) | ((b == a) & (jdx < idx))
    rank = jnp.sum(beats.astype(jnp.int32), axis=-1)
    return rank < k


def _gate_body(x_ref, gw_ref, bias_ref, comb_ref):
    x = x_ref[...]
    logits = lax.dot_general(x, gw_ref[...], (((1,), (1,)), ((), ())),
                             preferred_element_type=jnp.float32)
    scores = jax.nn.sigmoid(logits)
    sfc = scores + bias_ref[...]
    grow = lax.broadcasted_iota(jnp.int32, (NG, E), 0)
    gcol = lax.broadcasted_iota(jnp.int32, (NG, E), 1)
    M = (gcol // (E // NG) == grow).astype(jnp.float32)
    gm = lax.dot_general(sfc, M, (((1,), (1,)), ((), ())),
                         preferred_element_type=jnp.float32,
                         precision=lax.Precision.HIGHEST)
    gsel = _rank_lt(gm, TG)
    emaskf = lax.dot_general(gsel.astype(jnp.float32), M,
                             (((1,), (0,)), ((), ())),
                             preferred_element_type=jnp.float32)
    masked = jnp.where(emaskf > 0.5, sfc, -jnp.inf)
    esel = _rank_lt(masked, TOPK)
    w = jnp.where(esel, scores, 0.0)
    denom = jnp.sum(w, axis=-1, keepdims=True)
    comb_ref[...] = w / (denom + 1e-20) * RSF


def _experts_body(x_ref, w13_ref, w2_ref, comb_ref, out_ref, *, tb):
    e = pl.program_id(0)
    t = pl.program_id(1)
    x = x_ref[...]
    gu = lax.dot_general(x, w13_ref[0], (((1,), (1,)), ((), ())),
                         preferred_element_type=jnp.float32)
    g = gu[:, :DFF]
    u = gu[:, DFF:]
    act = g * jax.nn.sigmoid(g) * u
    eout = lax.dot_general(act, w2_ref[0], (((1,), (1,)), ((), ())),
                           preferred_element_type=jnp.float32)
    comb = comb_ref[pl.ds(t * tb, tb), :]
    lane = lax.broadcasted_iota(jnp.int32, comb.shape, 1)
    col = jnp.sum(jnp.where(lane == e, comb, 0.0), axis=1, keepdims=True)
    contrib = eout * col

    @pl.when(e == 0)
    def _init():
        out_ref[pl.ds(t * tb, tb), :] = contrib

    @pl.when(e > 0)
    def _acc():
        out_ref[pl.ds(t * tb, tb), :] += contrib


def _shared_body(x_ref, sgu_ref, sdn_ref, routed_ref, out_ref):
    x = x_ref[...]
    gu = lax.dot_general(x, sgu_ref[...], (((1,), (1,)), ((), ())),
                         preferred_element_type=jnp.float32)
    h = DFF * NSH
    g = gu[:, :h]
    u = gu[:, h:]
    act = g * jax.nn.sigmoid(g) * u
    sh = lax.dot_general(act, sdn_ref[...], (((1,), (1,)), ((), ())),
                         preferred_element_type=jnp.float32)
    out_ref[...] = routed_ref[...] + sh


@jax.jit
def kernel(hidden_states, gate_weight, e_score_correction_bias, w13, w2,
           shared_gate_up, shared_down):
    x = hidden_states
    bias2d = e_score_correction_bias.reshape(1, E)

    combine = pl.pallas_call(
        _gate_body,
        out_shape=jax.ShapeDtypeStruct((T, E), jnp.float32),
    )(x, gate_weight, bias2d)

    tb = 512
    nt = T // tb
    routed = pl.pallas_call(
        functools.partial(_experts_body, tb=tb),
        grid=(E, nt),
        in_specs=[
            pl.BlockSpec((tb, D), lambda e, t: (t, 0)),
            pl.BlockSpec((1, 2 * DFF, D), lambda e, t: (e, 0, 0)),
            pl.BlockSpec((1, D, DFF), lambda e, t: (e, 0, 0)),
            pl.BlockSpec((T, E), lambda e, t: (0, 0)),
        ],
        out_specs=pl.BlockSpec((T, D), lambda e, t: (0, 0)),
        out_shape=jax.ShapeDtypeStruct((T, D), jnp.float32),
    )(x, w13, w2, combine)

    tb2 = 512
    out = pl.pallas_call(
        _shared_body,
        grid=(T // tb2,),
        in_specs=[
            pl.BlockSpec((tb2, D), lambda t: (t, 0)),
            pl.BlockSpec((2 * DFF * NSH, D), lambda t: (0, 0)),
            pl.BlockSpec((D, DFF * NSH), lambda t: (0, 0)),
            pl.BlockSpec((tb2, D), lambda t: (t, 0)),
        ],
        out_specs=pl.BlockSpec((tb2, D), lambda t: (t, 0)),
        out_shape=jax.ShapeDtypeStruct((T, D), jnp.float32),
    )(x, shared_gate_up, shared_down, routed)
    return out
